# baseline (speedup 1.0000x reference)
import functools
import numpy as np
import jax
import jax.numpy as jnp
from jax import lax
from jax.experimental import pallas as pl
from jax.experimental.pallas import tpu as pltpu
from jax.experimental.pallas import tpu_sc as plsc

NN = 10000
T = 5
G = 64
NREAL = NN + T * G
NPAD = 10368
D = 128
BM = 384
NB = NPAD // BM
NNP = 10240


def _sim_body(tok_ref, x_ref, cross_ref, inner_ref):
    j = pl.program_id(0)
    tok = tok_ref[...]
    xb = x_ref[...]
    d = lax.dot_general(tok, xb, (((1,), (1,)), ((), ())),
                        preferred_element_type=jnp.float32)
    col = j * 1280 + lax.broadcasted_iota(jnp.int32, (8, 1280), 1)
    cross_ref[...] = jnp.where(col < NN, d, -1.0)

    @pl.when(j == 0)
    def _():
        i8 = lax.dot_general(tok, tok, (((1,), (1,)), ((), ())),
                             preferred_element_type=jnp.float32)
        inner_ref[...] = jnp.concatenate(
            [i8, jnp.zeros((8, 120), jnp.float32)], axis=1)


def _sim(tok_pad, x_padn):
    return pl.pallas_call(
        _sim_body,
        grid=(NNP // 1280,),
        in_specs=[
            pl.BlockSpec((8, 128), lambda j: (0, 0)),
            pl.BlockSpec((1280, 128), lambda j: (j, 0)),
        ],
        out_specs=[
            pl.BlockSpec((8, 1280), lambda j: (0, j)),
            pl.BlockSpec((8, 128), lambda j: (0, 0)),
        ],
        out_shape=[
            jax.ShapeDtypeStruct((8, NNP), jnp.float32),
            jax.ShapeDtypeStruct((8, 128), jnp.float32),
        ],
    )(tok_pad, x_padn)


def _deg_body(a_ref, dinv_ref, acc_ref):
    i = pl.program_id(0)
    j = pl.program_id(1)
    a = a_ref[...]
    s = lax.dot_general(a, jnp.ones((BM, 1), jnp.float32),
                        (((1,), (0,)), ((), ())),
                        preferred_element_type=jnp.float32)

    @pl.when(j == 0)
    def _():
        acc_ref[...] = jnp.zeros_like(acc_ref)

    acc_ref[...] += s

    @pl.when(j == NB - 1)
    def _():
        sub = jnp.sum(a[:, BM - (NPAD - NREAL):], axis=1, keepdims=True)
        deg = acc_ref[...] + 1.0 - sub
        dinv = jax.lax.rsqrt(jnp.maximum(deg, 1e-12))
        r = i * BM + lax.broadcasted_iota(jnp.int32, (BM, 1), 0)
        dinv_ref[...] = jnp.where(r < NREAL, dinv, 0.0)


def _deg(a_mat):
    return pl.pallas_call(
        _deg_body,
        grid=(NB, NB),
        in_specs=[pl.BlockSpec((BM, BM), lambda i, j: (i, j))],
        out_specs=pl.BlockSpec((BM, 1), lambda i, j: (i, 0)),
        out_shape=jax.ShapeDtypeStruct((NPAD, 1), jnp.float32),
        scratch_shapes=[pltpu.VMEM((BM, 1), jnp.float32)],
    )(a_mat)


def _y_body(h_ref, w_ref, dinv_ref, y_ref):
    y_ref[...] = lax.dot_general(
        h_ref[...], w_ref[...], (((1,), (0,)), ((), ())),
        preferred_element_type=jnp.float32) * dinv_ref[...]


def _y(h, w, dinv):
    return pl.pallas_call(
        _y_body,
        grid=(NB,),
        in_specs=[
            pl.BlockSpec((BM, D), lambda i: (i, 0)),
            pl.BlockSpec((D, D), lambda i: (0, 0)),
            pl.BlockSpec((BM, 1), lambda i: (i, 0)),
        ],
        out_specs=pl.BlockSpec((BM, D), lambda i: (i, 0)),
        out_shape=jax.ShapeDtypeStruct((NPAD, D), jnp.float32),
    )(h, w, dinv)


def _agg_body(a_ref, y_ref, dinv_ref, b_ref, out_ref, acc_ref, *, leaky):
    i = pl.program_id(0)
    j = pl.program_id(1)
    a = a_ref[...]
    yb = y_ref[pl.ds(j * BM, BM), :]

    @pl.when(j == 0)
    def _():
        acc_ref[...] = jnp.zeros_like(acc_ref)

    acc_ref[...] += lax.dot_general(
        a.astype(jnp.bfloat16), yb.astype(jnp.bfloat16),
        (((1,), (0,)), ((), ())), preferred_element_type=jnp.float32)

    @pl.when(j == NB - 1)
    def _():
        yi = y_ref[pl.ds(i * BM, BM), :]
        o = (acc_ref[...] + yi) * dinv_ref[...] + b_ref[...]
        if leaky:
            o = jnp.where(o >= 0, o, 0.01 * o)
        out_ref[...] = o


def _agg(a_mat, y, dinv, b2d, leaky):
    return pl.pallas_call(
        functools.partial(_agg_body, leaky=leaky),
        grid=(NB, NB),
        in_specs=[
            pl.BlockSpec((BM, BM), lambda i, j: (i, j)),
            pl.BlockSpec((NPAD, D), lambda i, j: (0, 0)),
            pl.BlockSpec((BM, 1), lambda i, j: (i, 0)),
            pl.BlockSpec((1, D), lambda i, j: (0, 0)),
        ],
        out_specs=pl.BlockSpec((BM, D), lambda i, j: (i, 0)),
        out_shape=jax.ShapeDtypeStruct((NPAD, D), jnp.float32),
        scratch_shapes=[pltpu.VMEM((BM, D), jnp.float32)],
    )(a_mat, y, dinv, b2d)


def _pool_body(emb_ref, bat_ref, wp_ref, bp_ref, out_ref, sum_ref, cnt_ref):
    i = pl.program_id(0)
    b = bat_ref[...]
    gids = lax.broadcasted_iota(jnp.int32, (64, BM), 0)
    p = jnp.where(gids == b, 1.0, 0.0)

    @pl.when(i == 0)
    def _():
        sum_ref[...] = jnp.zeros_like(sum_ref)
        cnt_ref[...] = jnp.zeros_like(cnt_ref)

    sum_ref[...] += lax.dot_general(p, emb_ref[...], (((1,), (0,)), ((), ())),
                                    preferred_element_type=jnp.float32)
    cnt_ref[...] += jnp.sum(p, axis=1, keepdims=True)

    @pl.when(i == NB - 1)
    def _():
        graph = sum_ref[...] / jnp.maximum(cnt_ref[...], 1.0)
        logits = lax.dot_general(graph, wp_ref[...], (((1,), (0,)), ((), ())),
                                 preferred_element_type=jnp.float32) + bp_ref[...]
        col = lax.broadcasted_iota(jnp.int32, (64, 128), 1)
        z = jnp.where(col < 2, logits, -1e30)
        m = jnp.max(z, axis=1, keepdims=True)
        e = jnp.where(col < 2, jnp.exp(z - m), 0.0)
        out_ref[...] = e / jnp.sum(e, axis=1, keepdims=True)


def _pool(emb, bat2d, wp_pad, bp_pad):
    return pl.pallas_call(
        _pool_body,
        grid=(NB,),
        in_specs=[
            pl.BlockSpec((BM, D), lambda i: (i, 0)),
            pl.BlockSpec((1, BM), lambda i: (0, i)),
            pl.BlockSpec((D, D), lambda i: (0, 0)),
            pl.BlockSpec((1, D), lambda i: (0, 0)),
        ],
        out_specs=pl.BlockSpec((64, 128), lambda i: (0, 0)),
        out_shape=jax.ShapeDtypeStruct((64, 128), jnp.float32),
        scratch_shapes=[pltpu.VMEM((64, D), jnp.float32),
                        pltpu.VMEM((64, 1), jnp.float32)],
    )(emb, bat2d, wp_pad, bp_pad)


FLAT = NPAD * NPAD
HALFR = NPAD // 2
STRIPE = FLAT // 32
ZCH = 13824
EPT = 20000
KROWS = 313
K2ROWS = 54
NNP16 = NNP // 16


def _build_body(esrc, edst, batchp, cross, innerf, innerr, innerc,
                a_out,
                zero_v, src_v, dst_v, codes, codes2, ones_v,
                batch_v, dot_v, ir_v, ic_v, if_v, zsem, ssem):
    c = lax.axis_index("c")
    s = lax.axis_index("s")
    lo = c * HALFR
    hi = lo + HALFR
    iota = lax.iota(jnp.int32, 16)
    dumpv = lo * NPAD + NREAL + iota

    def zfill(k, _):
        zero_v[pl.ds(k * 16, 16)] = jnp.zeros((16,), jnp.float32)
        return _
    lax.fori_loop(0, ZCH // 16, zfill, 0)
    def ofill(k, _):
        ones_v[k, pl.ds(0, 16)] = jnp.ones((16,), jnp.float32)
        ones_v[k, pl.ds(16, 16)] = jnp.ones((16,), jnp.float32)
        ones_v[k, pl.ds(32, 16)] = jnp.ones((16,), jnp.float32)
        ones_v[k, pl.ds(48, 16)] = jnp.ones((16,), jnp.float32)
        ones_v[k, pl.ds(64, 16)] = jnp.ones((16,), jnp.float32)
        ones_v[k, pl.ds(80, 16)] = jnp.ones((16,), jnp.float32)
        ones_v[k, pl.ds(96, 16)] = jnp.ones((16,), jnp.float32)
        ones_v[k, pl.ds(112, 16)] = jnp.ones((16,), jnp.float32)
        return _
    lax.fori_loop(0, KROWS, ofill, 0)

    base = c * (FLAT // 2) + s * STRIPE
    with jax.named_scope("zfire"):
        def zfire(k, _):
            pltpu.async_copy(
                zero_v, a_out.at[pl.ds(base + k * ZCH, ZCH)], zsem)
            return _
        lax.fori_loop(0, STRIPE // ZCH, zfire, 0)

    ebase = s * EPT
    for r in range(2):
        pltpu.sync_copy(esrc.at[pl.ds(ebase + r * 10000, 10000)], src_v)
        pltpu.sync_copy(edst.at[pl.ds(ebase + r * 10000, 10000)], dst_v)

        def estep(i, _):
            k = r * 625 + i
            sv = src_v[pl.ds(i * 16, 16)]
            dv = dst_v[pl.ds(i * 16, 16)]
            cf = jnp.where((dv >= lo) & (dv < hi), dv * NPAD + sv, dumpv)
            cb = jnp.where((sv >= lo) & (sv < hi), sv * NPAD + dv, dumpv)
            row = k >> 2
            col = (k & 3) * 32
            codes[row, pl.ds(col, 16)] = cf
            codes[row, pl.ds(col + 16, 16)] = cb
            return _
        lax.fori_loop(0, 625, estep, 0)
    for k in range(4):
        codes[KROWS - 1, pl.ds(64 + k * 16, 16)] = dumpv

    nbase = s * NNP16
    pltpu.sync_copy(batchp.at[pl.ds(nbase, NNP16)], batch_v)
    for t in range(T):
        pltpu.sync_copy(cross.at[t, pl.ds(nbase, NNP16)], dot_v)

        def cstep(k, _):
            n16 = nbase + k * 16 + iota
            b16 = batch_v[pl.ds(k * 16, 16)]
            dt = dot_v[pl.ds(k * 16, 16)]
            m = dt >= 0.0
            grow = NN + T * b16 + t
            cf = jnp.where(m & (grow >= lo) & (grow < hi),
                           grow * NPAD + n16, dumpv)
            cb = jnp.where(m & (n16 >= lo) & (n16 < hi),
                           n16 * NPAD + grow, dumpv)
            idx = k * T + t
            row = idx >> 3
            col = (idx & 7) * 16
            codes2[row, pl.ds(col, 16)] = cf
            codes2[25 + row, pl.ds(col, 16)] = cb
            return _
        lax.fori_loop(0, NNP16 // 16, cstep, 0)

    pltpu.sync_copy(innerr, ir_v)
    pltpu.sync_copy(innerc, ic_v)
    pltpu.sync_copy(innerf, if_v)
    for j in range(4):
        off = NN + T * (s * 4 + j)
        for h in range(2):
            rh = ir_v[pl.ds(h * 16, 16)]
            ch = ic_v[pl.ds(h * 16, 16)]
            fv = if_v[pl.ds(h * 16, 16)]
            row = off + rh
            cf = jnp.where((fv >= 0.0) & (row >= lo) & (row < hi),
                           row * NPAD + off + ch, dumpv)
            codes2[50, pl.ds((j * 2 + h) * 16, 16)] = cf

    for q in range(21):
        r16 = lo + s * 324 + q * 16 + iota
        cd = jnp.where(r16 < hi, r16 * (NPAD + 1), dumpv)
        codes2[51 + q // 8, pl.ds((q % 8) * 16, 16)] = cd
    for q in range(3):
        codes2[53, pl.ds(80 + q * 16, 16)] = dumpv

    with jax.named_scope("zwait"):
        def zdrain(k, _):
            pltpu.make_async_copy(
                zero_v, a_out.at[pl.ds(base, ZCH)], zsem).wait()
            return _
        lax.fori_loop(0, STRIPE // ZCH, zdrain, 0)
        plsc.subcore_barrier()

    with jax.named_scope("scatter"):
        ones_row = ones_v.at[0]
        def sfire(k, _):
            pltpu.async_copy(ones_row, a_out.at[codes.at[k]], ssem)
            return _
        lax.fori_loop(0, KROWS, sfire, 0)
        def sfire2(k, _):
            pltpu.async_copy(ones_row, a_out.at[codes2.at[k]], ssem)
            return _
        lax.fori_loop(0, K2ROWS, sfire2, 0)
        def sdrain(k, _):
            pltpu.make_async_copy(ones_row, a_out.at[codes.at[0]], ssem).wait()
            return _
        lax.fori_loop(0, KROWS + K2ROWS, sdrain, 0)


def _build_a(edge_index, batch, cross_dot, inner_dot):
    esrc = edge_index[0]
    edst = edge_index[1]
    batchp = jnp.concatenate([batch, jnp.zeros((NNP - NN,), jnp.int32)])
    inner_vals = inner_dot[:T, :T].reshape(T * T)
    innerf = jnp.concatenate([inner_vals, jnp.full((7,), -1.0, jnp.float32)])
    innerr = jnp.asarray(np.concatenate(
        [np.repeat(np.arange(T, dtype=np.int32), T), np.zeros(7, np.int32)]))
    innerc = jnp.asarray(np.concatenate(
        [np.tile(np.arange(T, dtype=np.int32), T), np.zeros(7, np.int32)]))

    mesh = plsc.VectorSubcoreMesh(core_axis_name="c", subcore_axis_name="s")
    build = pl.kernel(
        _build_body,
        out_type=jax.ShapeDtypeStruct((FLAT,), jnp.float32),
        mesh=mesh,
        scratch_types=[
            pltpu.VMEM((ZCH,), jnp.float32),
            pltpu.VMEM((10000,), jnp.int32),
            pltpu.VMEM((10000,), jnp.int32),
            pltpu.VMEM((KROWS, 128), jnp.int32),
            pltpu.VMEM((K2ROWS, 128), jnp.int32),
            pltpu.VMEM((KROWS, 128), jnp.float32),
            pltpu.VMEM((NNP16,), jnp.int32),
            pltpu.VMEM((NNP16,), jnp.float32),
            pltpu.VMEM((32,), jnp.int32),
            pltpu.VMEM((32,), jnp.int32),
            pltpu.VMEM((32,), jnp.float32),
            pltpu.SemaphoreType.DMA,
            pltpu.SemaphoreType.DMA,
        ],
    )
    a_flat = build(esrc, edst, batchp, cross_dot, innerf, innerr, innerc)
    return a_flat.reshape(NPAD, NPAD)


def kernel(x, edge_index, batch, num_graphs, token_x, W1, b1, W2, b2, Wp, bp):
    del num_graphs
    f32 = jnp.float32
    tok_pad = jnp.concatenate([token_x, jnp.zeros((3, D), f32)], axis=0)
    x_padn = jnp.concatenate([x, jnp.zeros((NNP - NN, D), f32)], axis=0)
    cross_dot, inner_dot = _sim(tok_pad, x_padn)

    a_mat = _build_a(edge_index, batch, cross_dot, inner_dot)

    dinv = _deg(a_mat)

    x_aug = jnp.concatenate(
        [x, jnp.tile(token_x, (G, 1)), jnp.zeros((NPAD - NREAL, D), f32)],
        axis=0)
    b1_2d = b1.reshape(1, D)
    b2_2d = b2.reshape(1, D)

    y1 = _y(x_aug, W1, dinv)
    h1 = _agg(a_mat, y1, dinv, b1_2d, leaky=True)
    y2 = _y(h1, W2, dinv)
    emb = _agg(a_mat, y2, dinv, b2_2d, leaky=False)

    token_batch = np.repeat(np.arange(G, dtype=np.int32), T)
    pad_batch = np.full((NPAD - NREAL,), -1, np.int32)
    bat2d = jnp.concatenate(
        [batch, jnp.asarray(token_batch), jnp.asarray(pad_batch)]
    ).reshape(1, NPAD)
    wp_pad = jnp.concatenate([Wp, jnp.zeros((D, D - 2), f32)], axis=1)
    bp_pad = jnp.concatenate([bp, jnp.zeros((D - 2,), f32)]).reshape(1, D)

    out = _pool(emb, bat2d, wp_pad, bp_pad)
    return out[:, :2]


# speedup vs baseline: 25.0334x; 25.0334x over previous
import functools
import numpy as np
import jax
import jax.numpy as jnp
from jax import lax
from jax.experimental import pallas as pl
from jax.experimental.pallas import tpu as pltpu
from jax.experimental.pallas import tpu_sc as plsc

NN = 10000
T = 5
G = 64
NREAL = NN + T * G
NPAD = 10368
D = 128
BM = 384
NB = NPAD // BM
NNP = 10240


def _sim_body(tok_ref, x_ref, cross_ref, inner_ref):
    j = pl.program_id(0)
    tok = tok_ref[...]
    xb = x_ref[...]
    d = lax.dot_general(tok, xb, (((1,), (1,)), ((), ())),
                        preferred_element_type=jnp.float32)
    col = j * 1280 + lax.broadcasted_iota(jnp.int32, (8, 1280), 1)
    cross_ref[...] = jnp.where(col < NN, d, -1.0)

    @pl.when(j == 0)
    def _():
        i8 = lax.dot_general(tok, tok, (((1,), (1,)), ((), ())),
                             preferred_element_type=jnp.float32)
        inner_ref[...] = jnp.concatenate(
            [i8, jnp.zeros((8, 120), jnp.float32)], axis=1)


def _sim(tok_pad, x_padn):
    return pl.pallas_call(
        _sim_body,
        grid=(NNP // 1280,),
        in_specs=[
            pl.BlockSpec((8, 128), lambda j: (0, 0)),
            pl.BlockSpec((1280, 128), lambda j: (j, 0)),
        ],
        out_specs=[
            pl.BlockSpec((8, 1280), lambda j: (0, j)),
            pl.BlockSpec((8, 128), lambda j: (0, 0)),
        ],
        out_shape=[
            jax.ShapeDtypeStruct((8, NNP), jnp.float32),
            jax.ShapeDtypeStruct((8, 128), jnp.float32),
        ],
    )(tok_pad, x_padn)


def _deg_body(a_ref, dinv_ref, abf_ref, acc_ref):
    i = pl.program_id(0)
    j = pl.program_id(1)
    a4 = a_ref[...]
    abf_ref[...] = a4.astype(jnp.bfloat16)
    ones = jnp.ones((128, 1), jnp.float32)
    s = jnp.zeros((BM, 1), jnp.float32)
    for c7 in range(3):
        a2 = a4[:, c7].reshape(BM, 128)
        s += lax.dot_general(a2, ones, (((1,), (0,)), ((), ())),
                             preferred_element_type=jnp.float32)

    @pl.when(j == 0)
    def _():
        acc_ref[...] = jnp.zeros_like(acc_ref)

    acc_ref[...] += s

    @pl.when(j == NB - 1)
    def _():
        a2 = a4[:, 2].reshape(BM, 128)
        sub = jnp.sum(a2[:, 128 - (NPAD - NREAL):], axis=1, keepdims=True)
        deg = acc_ref[...] + 1.0 - sub
        dinv = jax.lax.rsqrt(jnp.maximum(deg, 1e-12))
        r = i * BM + lax.broadcasted_iota(jnp.int32, (BM, 1), 0)
        dinv_ref[...] = jnp.where(r < NREAL, dinv, 0.0)


def _deg(a4d):
    return pl.pallas_call(
        _deg_body,
        grid=(NB, NB),
        in_specs=[pl.BlockSpec((48, 3, 8, 128), lambda i, j: (i, j, 0, 0))],
        out_specs=[
            pl.BlockSpec((BM, 1), lambda i, j: (i, 0)),
            pl.BlockSpec((48, 3, 8, 128), lambda i, j: (i, j, 0, 0)),
        ],
        out_shape=[
            jax.ShapeDtypeStruct((NPAD, 1), jnp.float32),
            jax.ShapeDtypeStruct(
                (NPAD // 8, NPAD // 128, 8, 128), jnp.bfloat16),
        ],
        scratch_shapes=[pltpu.VMEM((BM, 1), jnp.float32)],
    )(a4d)


def _agg_body(a_ref, h_ref, w_ref, dinv_ref, b_ref, out_ref,
              acc_ref, y_ref, *, leaky):
    i = pl.program_id(0)
    j = pl.program_id(1)
    a4 = a_ref[...]

    @pl.when(i == 0)
    def _():
        hb = h_ref[pl.ds(j * BM, BM), :]
        db = dinv_ref[pl.ds(j * BM, BM), :]
        y_ref[pl.ds(j * BM, BM), :] = lax.dot_general(
            hb, w_ref[...], (((1,), (0,)), ((), ())),
            preferred_element_type=jnp.float32) * db

    @pl.when(j == 0)
    def _():
        acc_ref[...] = jnp.zeros_like(acc_ref)

    acc = jnp.zeros((BM, D), jnp.float32)
    for c7 in range(3):
        a2 = a4[:, c7].reshape(BM, 128)
        yb = y_ref[pl.ds((j * 3 + c7) * 128, 128), :]
        acc += lax.dot_general(
            a2, yb.astype(jnp.bfloat16),
            (((1,), (0,)), ((), ())), preferred_element_type=jnp.float32)
    acc_ref[...] += acc

    @pl.when(j == NB - 1)
    def _():
        yi = y_ref[pl.ds(i * BM, BM), :]
        o = (acc_ref[...] + yi) * dinv_ref[pl.ds(i * BM, BM), :] + b_ref[...]
        if leaky:
            o = jnp.where(o >= 0, o, 0.01 * o)
        out_ref[...] = o


def _agg(a_mat, h, w, dinv, b2d, leaky):
    return pl.pallas_call(
        functools.partial(_agg_body, leaky=leaky),
        grid=(NB, NB),
        in_specs=[
            pl.BlockSpec((48, 3, 8, 128), lambda i, j: (i, j, 0, 0)),
            pl.BlockSpec((NPAD, D), lambda i, j: (0, 0)),
            pl.BlockSpec((D, D), lambda i, j: (0, 0)),
            pl.BlockSpec((NPAD, 1), lambda i, j: (0, 0)),
            pl.BlockSpec((1, D), lambda i, j: (0, 0)),
        ],
        out_specs=pl.BlockSpec((BM, D), lambda i, j: (i, 0)),
        out_shape=jax.ShapeDtypeStruct((NPAD, D), jnp.float32),
        scratch_shapes=[pltpu.VMEM((BM, D), jnp.float32),
                        pltpu.VMEM((NPAD, D), jnp.float32)],
    )(a_mat, h, w, dinv, b2d)


def _agg_pool_body(a_ref, h_ref, w_ref, dinv_ref, b_ref, bat_ref, wp_ref,
                   bp_ref, out_ref, pool_ref, acc_ref, y_ref, sum_ref,
                   cnt_ref):
    i = pl.program_id(0)
    j = pl.program_id(1)
    a4 = a_ref[...]

    @pl.when(i == 0)
    def _():
        hb = h_ref[pl.ds(j * BM, BM), :]
        db = dinv_ref[pl.ds(j * BM, BM), :]
        y_ref[pl.ds(j * BM, BM), :] = lax.dot_general(
            hb, w_ref[...], (((1,), (0,)), ((), ())),
            preferred_element_type=jnp.float32) * db

    @pl.when(j == 0)
    def _():
        acc_ref[...] = jnp.zeros_like(acc_ref)

    acc = jnp.zeros((BM, D), jnp.float32)
    for c7 in range(3):
        a2 = a4[:, c7].reshape(BM, 128)
        yb = y_ref[pl.ds((j * 3 + c7) * 128, 128), :]
        acc += lax.dot_general(
            a2, yb.astype(jnp.bfloat16),
            (((1,), (0,)), ((), ())), preferred_element_type=jnp.float32)
    acc_ref[...] += acc

    @pl.when(j == NB - 1)
    def _():
        yi = y_ref[pl.ds(i * BM, BM), :]
        o = (acc_ref[...] + yi) * dinv_ref[pl.ds(i * BM, BM), :] + b_ref[...]
        out_ref[...] = o

        @pl.when(i == 0)
        def _():
            sum_ref[...] = jnp.zeros_like(sum_ref)
            cnt_ref[...] = jnp.zeros_like(cnt_ref)

        bb = bat_ref[...]
        gids = lax.broadcasted_iota(jnp.int32, (64, BM), 0)
        pm = jnp.where(gids == bb, 1.0, 0.0)
        sum_ref[...] += lax.dot_general(
            pm, o, (((1,), (0,)), ((), ())),
            preferred_element_type=jnp.float32)
        cnt_ref[...] += jnp.sum(pm, axis=1, keepdims=True)

        @pl.when(i == NB - 1)
        def _():
            graph = sum_ref[...] / jnp.maximum(cnt_ref[...], 1.0)
            logits = lax.dot_general(
                graph, wp_ref[...], (((1,), (0,)), ((), ())),
                preferred_element_type=jnp.float32) + bp_ref[...]
            col = lax.broadcasted_iota(jnp.int32, (64, 128), 1)
            z = jnp.where(col < 2, logits, -1e30)
            m = jnp.max(z, axis=1, keepdims=True)
            e = jnp.where(col < 2, jnp.exp(z - m), 0.0)
            pool_ref[...] = e / jnp.sum(e, axis=1, keepdims=True)


def _agg_pool(a4d, h, w, dinv, b2d, bat2d, wp_pad, bp_pad):
    return pl.pallas_call(
        _agg_pool_body,
        grid=(NB, NB),
        in_specs=[
            pl.BlockSpec((48, 3, 8, 128), lambda i, j: (i, j, 0, 0)),
            pl.BlockSpec((NPAD, D), lambda i, j: (0, 0)),
            pl.BlockSpec((D, D), lambda i, j: (0, 0)),
            pl.BlockSpec((NPAD, 1), lambda i, j: (0, 0)),
            pl.BlockSpec((1, D), lambda i, j: (0, 0)),
            pl.BlockSpec((1, BM), lambda i, j: (0, i)),
            pl.BlockSpec((D, D), lambda i, j: (0, 0)),
            pl.BlockSpec((1, D), lambda i, j: (0, 0)),
        ],
        out_specs=[
            pl.BlockSpec((BM, D), lambda i, j: (i, 0)),
            pl.BlockSpec((64, 128), lambda i, j: (0, 0)),
        ],
        out_shape=[
            jax.ShapeDtypeStruct((NPAD, D), jnp.float32),
            jax.ShapeDtypeStruct((64, 128), jnp.float32),
        ],
        scratch_shapes=[pltpu.VMEM((BM, D), jnp.float32),
                        pltpu.VMEM((NPAD, D), jnp.float32),
                        pltpu.VMEM((64, D), jnp.float32),
                        pltpu.VMEM((64, 1), jnp.float32)],
    )(a4d, h, w, dinv, b2d, bat2d, wp_pad, bp_pad)


FLAT = NPAD * NPAD
EPT = 10000
KROWS = 157
K2ROWS = 29
NNP32 = NNP // 32
DROWS = NPAD // 32


def _build_body(esrc, edst, batchp, cross, innerf, innerr, innerc, a_out,
                src_v, dst_v, codes, codes2, ones_v,
                batch_v, dot_v, ir_v, ic_v, if_v, ssem):
    c = lax.axis_index("c")
    s = lax.axis_index("s")
    w = c * 16 + s
    iota = lax.iota(jnp.int32, 16)
    dbase = w * DROWS

    def code(r, cc):
        return (((r >> 3) * (NPAD // 128) + (cc >> 7)) * 1024
                + ((r & 7) << 7) + (cc & 127))

    def dump(r):
        return code(dbase + r, NREAL + iota)

    def ofill(k, _):
        for q in range(8):
            ones_v[k, pl.ds(q * 16, 16)] = jnp.ones((16,), jnp.float32)
        return _
    lax.fori_loop(0, KROWS, ofill, 0)

    pltpu.sync_copy(esrc.at[pl.ds(w * EPT, EPT)], src_v)
    pltpu.sync_copy(edst.at[pl.ds(w * EPT, EPT)], dst_v)

    def estep(i, _):
        sv = src_v[pl.ds(i * 16, 16)]
        dv = dst_v[pl.ds(i * 16, 16)]
        row = i >> 2
        col = (i & 3) * 32
        codes[row, pl.ds(col, 16)] = code(dv, sv)
        codes[row, pl.ds(col + 16, 16)] = code(sv, dv)
        return _
    lax.fori_loop(0, EPT // 16, estep, 0)
    for q in range(6):
        codes[KROWS - 1, pl.ds(32 + q * 16, 16)] = dump(q)

    nbase = w * NNP32
    pltpu.sync_copy(batchp.at[pl.ds(nbase, NNP32)], batch_v)
    for t in range(T):
        pltpu.sync_copy(cross.at[pl.ds(t * NNP + nbase, NNP32)], dot_v)

        def cstep(k, _):
            n16 = nbase + k * 16 + iota
            b16 = batch_v[pl.ds(k * 16, 16)]
            dt = dot_v[pl.ds(k * 16, 16)]
            m = dt >= 0.0
            grow = NN + T * b16 + t
            q = (k * T + t) * 2
            dv_ = dump(q)
            cf = jnp.where(m, code(grow, n16), dv_)
            cb = jnp.where(m, code(n16, grow), dump(q + 1))
            row = q >> 3
            col = (q & 7) * 16
            codes2[row, pl.ds(col, 16)] = cf
            row2 = (q + 1) >> 3
            col2 = ((q + 1) & 7) * 16
            codes2[row2, pl.ds(col2, 16)] = cb
            return _
        lax.fori_loop(0, NNP32 // 16, cstep, 0)

    for q in range(21):
        r16 = dbase + q * 16 + iota
        cd = jnp.where(r16 < NPAD, code(r16, r16), dump(q))
        codes2[25 + q // 8, pl.ds((q % 8) * 16, 16)] = cd

    pltpu.sync_copy(innerr, ir_v)
    pltpu.sync_copy(innerc, ic_v)
    pltpu.sync_copy(innerf, if_v)
    for j in range(2):
        off = NN + T * (w * 2 + j)
        for h in range(2):
            rh = ir_v[pl.ds(h * 16, 16)]
            ch = ic_v[pl.ds(h * 16, 16)]
            fv = if_v[pl.ds(h * 16, 16)]
            rowv = off + rh
            m = j * 2 + h
            flat = 3536 + 16 * m
            cd = jnp.where(fv >= 0.0, code(rowv, off + ch), dump(32 + m))
            codes2[flat >> 7, pl.ds(flat & 127, 16)] = cd
    for q in range(7):
        flat = 3600 + 16 * q
        codes2[flat >> 7, pl.ds(flat & 127, 16)] = dump(40 + q)

    ones_row = ones_v.at[0]
    def sfire(k, _):
        pltpu.async_copy(ones_row, a_out.at[codes.at[k]], ssem)
        return _
    lax.fori_loop(0, KROWS, sfire, 0)
    def sfire2(k, _):
        pltpu.async_copy(ones_row, a_out.at[codes2.at[k]], ssem)
        return _
    lax.fori_loop(0, K2ROWS, sfire2, 0)
    def sdrain(k, _):
        pltpu.make_async_copy(ones_row, a_out.at[codes.at[0]], ssem).wait()
        return _
    lax.fori_loop(0, KROWS + K2ROWS, sdrain, 0)


def _build_a(edge_index, batch, cross_dot, inner_dot):
    esrc = edge_index[0]
    edst = edge_index[1]
    batchp = jnp.concatenate([batch, jnp.zeros((NNP - NN,), jnp.int32)])
    inner_vals = inner_dot[:T, :T].reshape(T * T)
    innerf = jnp.concatenate([inner_vals, jnp.full((7,), -1.0, jnp.float32)])
    innerr = jnp.asarray(np.concatenate(
        [np.repeat(np.arange(T, dtype=np.int32), T), np.zeros(7, np.int32)]))
    innerc = jnp.asarray(np.concatenate(
        [np.tile(np.arange(T, dtype=np.int32), T), np.zeros(7, np.int32)]))

    mesh = plsc.VectorSubcoreMesh(core_axis_name="c", subcore_axis_name="s")
    build = pl.kernel(
        _build_body,
        out_type=(),
        mesh=mesh,
        scratch_types=[
            pltpu.VMEM((EPT,), jnp.int32),
            pltpu.VMEM((EPT,), jnp.int32),
            pltpu.VMEM((KROWS, 128), jnp.int32),
            pltpu.VMEM((K2ROWS, 128), jnp.int32),
            pltpu.VMEM((KROWS, 128), jnp.float32),
            pltpu.VMEM((NNP32,), jnp.int32),
            pltpu.VMEM((NNP32,), jnp.float32),
            pltpu.VMEM((32,), jnp.int32),
            pltpu.VMEM((32,), jnp.int32),
            pltpu.VMEM((32,), jnp.float32),
            pltpu.SemaphoreType.DMA,
        ],
    )
    crossf = cross_dot.reshape(8 * NNP)
    a_ref = jax.new_ref(jnp.zeros((FLAT,), jnp.float32))
    build(esrc, edst, batchp, crossf, innerf, innerr, innerc, a_ref)
    return a_ref[...].reshape(NPAD // 8, NPAD // 128, 8, 128)


def kernel(x, edge_index, batch, num_graphs, token_x, W1, b1, W2, b2, Wp, bp):
    del num_graphs
    f32 = jnp.float32
    tok_pad = jnp.concatenate([token_x, jnp.zeros((3, D), f32)], axis=0)
    x_padn = jnp.concatenate([x, jnp.zeros((NNP - NN, D), f32)], axis=0)
    cross_dot, inner_dot = _sim(tok_pad, x_padn)

    a_mat = _build_a(edge_index, batch, cross_dot, inner_dot)

    dinv, a_bf = _deg(a_mat)

    x_aug = jnp.concatenate(
        [x, jnp.tile(token_x, (G, 1)), jnp.zeros((NPAD - NREAL, D), f32)],
        axis=0)
    b1_2d = b1.reshape(1, D)
    b2_2d = b2.reshape(1, D)

    token_batch = np.repeat(np.arange(G, dtype=np.int32), T)
    pad_batch = np.full((NPAD - NREAL,), -1, np.int32)
    bat2d = jnp.concatenate(
        [batch, jnp.asarray(token_batch), jnp.asarray(pad_batch)]
    ).reshape(1, NPAD)
    wp_pad = jnp.concatenate([Wp, jnp.zeros((D, D - 2), f32)], axis=1)
    bp_pad = jnp.concatenate([bp, jnp.zeros((D - 2,), f32)]).reshape(1, D)

    h1 = _agg(a_bf, x_aug, W1, dinv, b1_2d, leaky=True)
    _, out = _agg_pool(a_bf, h1, W2, dinv, b2_2d, bat2d, wp_pad, bp_pad)
    return out[:, :2]
